# baseline (speedup 1.0000x reference)
import jax
import jax.numpy as jnp
from jax.experimental import pallas as pl
from jax.experimental.pallas import tpu as pltpu

BLOCK_K = 32


def _body(ht_ref, w_ref, b_ref, m_ref, o_ref):
    j = pl.program_id(0)
    acc = jax.lax.dot_general(
        ht_ref[...], w_ref[...],
        dimension_numbers=(((0,), (0,)), ((), ())),
        preferred_element_type=jnp.float32,
    )

    @pl.when(j == 0)
    def _init():
        o_ref[...] = acc + b_ref[...] + (1.0 - m_ref[...]) * -1000000000.0

    @pl.when(j != 0)
    def _accum():
        o_ref[...] += acc


def kernel(hidden, W, b, mask):
    B, H = hidden.shape
    V = W.shape[1]
    b2 = b.reshape(1, V)
    m2 = mask.reshape(1, V)
    ht = hidden.T
    grid = (H // BLOCK_K,)
    return pl.pallas_call(
        _body,
        grid=grid,
        in_specs=[
            pl.BlockSpec((BLOCK_K, B), lambda j: (j, 0)),
            pl.BlockSpec((BLOCK_K, V), lambda j: (j, 0)),
            pl.BlockSpec((1, V), lambda j: (0, 0)),
            pl.BlockSpec((1, V), lambda j: (0, 0)),
        ],
        out_specs=pl.BlockSpec((B, V), lambda j: (0, 0)),
        out_shape=jax.ShapeDtypeStruct((B, V), jnp.float32),
        compiler_params=pltpu.CompilerParams(
            dimension_semantics=("arbitrary",),
        ),
    )(ht, W, b2, m2)


# speedup vs baseline: 3.7140x; 3.7140x over previous
import jax
import jax.numpy as jnp
from jax.experimental import pallas as pl
from jax.experimental.pallas import tpu as pltpu

BLOCK_N = 4096


def _body(h_ref, wt_ref, b_ref, m_ref, o_ref):
    acc = jax.lax.dot_general(
        h_ref[...], wt_ref[...],
        dimension_numbers=(((1,), (1,)), ((), ())),
        preferred_element_type=jnp.float32,
    )
    o_ref[...] = acc + b_ref[...] + (1.0 - m_ref[...]) * -1000000000.0


def kernel(hidden, W, b, mask):
    B, H = hidden.shape
    V = W.shape[1]
    wt = W.T
    b2 = b.reshape(1, V)
    m2 = mask.reshape(1, V)
    grid = (pl.cdiv(V, BLOCK_N),)
    return pl.pallas_call(
        _body,
        grid=grid,
        in_specs=[
            pl.BlockSpec((B, H), lambda j: (0, 0)),
            pl.BlockSpec((BLOCK_N, H), lambda j: (j, 0)),
            pl.BlockSpec((1, BLOCK_N), lambda j: (0, j)),
            pl.BlockSpec((1, BLOCK_N), lambda j: (0, j)),
        ],
        out_specs=pl.BlockSpec((B, BLOCK_N), lambda j: (0, j)),
        out_shape=jax.ShapeDtypeStruct((B, V), jnp.float32),
        compiler_params=pltpu.CompilerParams(
            dimension_semantics=("parallel",),
        ),
    )(hidden, wt, b2, m2)
